# Initial kernel scaffold; baseline (speedup 1.0000x reference)
import functools

import jax
import jax.numpy as jnp
from jax import lax
from jax.experimental import pallas as pl
from jax.experimental.pallas import tpu as pltpu
from jax.experimental.pallas import tpu_sc as plsc

B = 16384
HIST = 50
D_UID = 64
D_SP = 32
D_AR = 32
DNN_H = 256
DNN_OUT = 128

NC, NS = 2, 16
NW = NC * NS
BPW = B // NW
CH = 128
AR_PER_W = BPW * HIST

_mesh = plsc.VectorSubcoreMesh(core_axis_name="c", subcore_axis_name="s")


@functools.partial(
    pl.kernel,
    mesh=_mesh,
    out_type=[
        jax.ShapeDtypeStruct((B, D_UID), jnp.float32),
        jax.ShapeDtypeStruct((B, D_SP), jnp.float32),
        jax.ShapeDtypeStruct((B, D_SP), jnp.float32),
        jax.ShapeDtypeStruct((B, D_SP), jnp.float32),
        jax.ShapeDtypeStruct((B, D_SP), jnp.float32),
        jax.ShapeDtypeStruct((B * HIST, D_AR), jnp.float32),
        jax.ShapeDtypeStruct((B * HIST, D_AR), jnp.float32),
    ],
    scratch_types=[
        pltpu.VMEM((CH,), jnp.int32),
        pltpu.VMEM((CH, D_UID), jnp.float32),
        pltpu.VMEM((CH, D_SP), jnp.float32),
    ],
)
def _sc_gather(seq, s0, s1, s2, s3, a0f, a1f,
               tu, t0, t1, t2, t3, ta0, ta1,
               uid_o, sp0_o, sp1_o, sp2_o, sp3_o, g0_o, g1_o,
               idx_v, rows64_v, rows32_v):
    wid = lax.axis_index("s") * NC + lax.axis_index("c")
    base = wid * BPW

    @pl.loop(0, BPW // CH)
    def _(c):
        off = base + c * CH
        pltpu.sync_copy(seq.at[pl.ds(off, CH)], idx_v)
        pltpu.sync_copy(tu.at[idx_v], rows64_v)
        pltpu.sync_copy(rows64_v, uid_o.at[pl.ds(off, CH)])

    for sidx, stab, so in ((s0, t0, sp0_o), (s1, t1, sp1_o),
                           (s2, t2, sp2_o), (s3, t3, sp3_o)):
        @pl.loop(0, BPW // CH)
        def _(c, sidx=sidx, stab=stab, so=so):
            off = base + c * CH
            pltpu.sync_copy(sidx.at[pl.ds(off, CH)], idx_v)
            pltpu.sync_copy(stab.at[idx_v], rows32_v)
            pltpu.sync_copy(rows32_v, so.at[pl.ds(off, CH)])

    for af, atab, go in ((a0f, ta0, g0_o), (a1f, ta1, g1_o)):
        @pl.loop(0, AR_PER_W // CH)
        def _(c, af=af, atab=atab, go=go):
            off = base * HIST + c * CH
            pltpu.sync_copy(af.at[pl.ds(off, CH)], idx_v)
            pltpu.sync_copy(atab.at[idx_v], rows32_v)
            pltpu.sync_copy(rows32_v, go.at[pl.ds(off, CH)])


ROWS = 512


def _mlp_body(uid, sp0, sp1, sp2, sp3, g0, g1, dn, w1, w1d, b1, w2, b2, out):
    acc = jnp.dot(uid[...], w1[0:64, :], preferred_element_type=jnp.float32)
    acc += jnp.dot(sp0[...], w1[64:96, :], preferred_element_type=jnp.float32)
    acc += jnp.dot(sp1[...], w1[96:128, :], preferred_element_type=jnp.float32)
    acc += jnp.dot(sp2[...], w1[128:160, :], preferred_element_type=jnp.float32)
    acc += jnp.dot(sp3[...], w1[160:192, :], preferred_element_type=jnp.float32)
    acc += jnp.dot(g0[...].sum(axis=1), w1[192:224, :],
                   preferred_element_type=jnp.float32)
    acc += jnp.dot(g1[...].sum(axis=1), w1[224:256, :],
                   preferred_element_type=jnp.float32)
    acc += jnp.dot(dn[...], w1d[...], preferred_element_type=jnp.float32)
    h = jnp.maximum(acc + b1[...], 0.0)
    out[...] = jnp.dot(h, w2[...], preferred_element_type=jnp.float32) + b2[...]


def _mlp(uid, sp0, sp1, sp2, sp3, g0, g1, dn, w1, w1d, b1, w2, b2):
    grid = (B // ROWS,)
    row_spec = lambda d: pl.BlockSpec((ROWS, d), lambda i: (i, 0))
    full_spec = lambda a, b: pl.BlockSpec((a, b), lambda i: (0, 0))
    return pl.pallas_call(
        _mlp_body,
        grid=grid,
        in_specs=[
            row_spec(D_UID),
            row_spec(D_SP), row_spec(D_SP), row_spec(D_SP), row_spec(D_SP),
            pl.BlockSpec((ROWS, HIST, D_AR), lambda i: (i, 0, 0)),
            pl.BlockSpec((ROWS, HIST, D_AR), lambda i: (i, 0, 0)),
            row_spec(8),
            full_spec(256, DNN_H),
            full_spec(8, DNN_H),
            full_spec(1, DNN_H),
            full_spec(DNN_H, DNN_OUT),
            full_spec(1, DNN_OUT),
        ],
        out_specs=pl.BlockSpec((ROWS, DNN_OUT), lambda i: (i, 0)),
        out_shape=jax.ShapeDtypeStruct((B, DNN_OUT), jnp.float32),
        compiler_params=pltpu.CompilerParams(
            dimension_semantics=("arbitrary",)),
    )(uid, sp0, sp1, sp2, sp3, g0, g1, dn, w1, w1d, b1, w2, b2)


def kernel(seq_id, sparse_0, sparse_1, sparse_2, sparse_3, array_0, array_1,
           dense_0, dense_1, dense_2, emb_user_id,
           emb_sparse_0, emb_sparse_1, emb_sparse_2, emb_sparse_3,
           emb_array_0, emb_array_1, W1, b1, W2, b2):
    a0f = array_0.reshape(-1)
    a1f = array_1.reshape(-1)
    uid_g, sp0_g, sp1_g, sp2_g, sp3_g, g0, g1 = _sc_gather(
        seq_id, sparse_0, sparse_1, sparse_2, sparse_3, a0f, a1f,
        emb_user_id, emb_sparse_0, emb_sparse_1, emb_sparse_2, emb_sparse_3,
        emb_array_0, emb_array_1)
    dn = jnp.pad(jnp.stack([dense_0, dense_1, dense_2], axis=1),
                 ((0, 0), (0, 5)))
    w1main = W1[:256]
    w1d = jnp.pad(W1[256:], ((0, 5), (0, 0)))
    return _mlp(uid_g, sp0_g, sp1_g, sp2_g, sp3_g,
                g0.reshape(B, HIST, D_AR), g1.reshape(B, HIST, D_AR),
                dn, w1main, w1d, b1.reshape(1, -1), W2, b2.reshape(1, -1))


# speedup vs baseline: 5.0459x; 5.0459x over previous
import functools

import jax
import jax.numpy as jnp
from jax import lax
from jax.experimental import pallas as pl
from jax.experimental.pallas import tpu as pltpu
from jax.experimental.pallas import tpu_sc as plsc

B = 16384
HIST = 50
D_UID = 64
D_SP = 32
D_AR = 32
DNN_H = 256
DNN_OUT = 128

NC, NS = 2, 16
NW = NC * NS
BPW = B // NW
CH = 128
AR_PER_W = BPW * HIST
N_ACH = AR_PER_W // CH

_mesh = plsc.VectorSubcoreMesh(core_axis_name="c", subcore_axis_name="s")


@functools.partial(
    pl.kernel,
    mesh=_mesh,
    compiler_params=pltpu.CompilerParams(use_tc_tiling_on_sc=False),
    out_type=[
        jax.ShapeDtypeStruct((B, D_UID), jnp.float32),
        jax.ShapeDtypeStruct((B, D_SP), jnp.float32),
        jax.ShapeDtypeStruct((B, D_SP), jnp.float32),
        jax.ShapeDtypeStruct((B, D_SP), jnp.float32),
        jax.ShapeDtypeStruct((B, D_SP), jnp.float32),
        jax.ShapeDtypeStruct((B, D_AR), jnp.float32),
        jax.ShapeDtypeStruct((B, D_AR), jnp.float32),
    ],
    scratch_types=[
        pltpu.VMEM((CH,), jnp.int32),
        pltpu.VMEM((CH, D_UID), jnp.float32),
        pltpu.VMEM((CH, D_SP), jnp.float32),
        pltpu.VMEM((N_ACH, CH), jnp.int32),
        pltpu.VMEM_SHARED((NS * BPW, D_AR), jnp.float32),
    ],
)
def _sc_gather(seq, s0, s1, s2, s3, a0f, a1f, seg, zrows,
               tu, t0, t1, t2, t3, ta0, ta1,
               uid_o, sp0_o, sp1_o, sp2_o, sp3_o, p0_o, p1_o,
               idx_v, rows64_v, rows32_v, seg_v, pooled_sh):
    sid = lax.axis_index("s")
    wid = sid * NC + lax.axis_index("c")
    base = wid * BPW

    pltpu.sync_copy(seg.at[sid], seg_v)

    @pl.loop(0, BPW // CH)
    def _(c):
        off = base + c * CH
        pltpu.sync_copy(seq.at[pl.ds(off, CH)], idx_v)
        pltpu.sync_copy(tu.at[idx_v], rows64_v)
        pltpu.sync_copy(rows64_v, uid_o.at[pl.ds(off, CH)])

    for sidx, stab, so in ((s0, t0, sp0_o), (s1, t1, sp1_o),
                           (s2, t2, sp2_o), (s3, t3, sp3_o)):
        @pl.loop(0, BPW // CH)
        def _(c, sidx=sidx, stab=stab, so=so):
            off = base + c * CH
            pltpu.sync_copy(sidx.at[pl.ds(off, CH)], idx_v)
            pltpu.sync_copy(stab.at[idx_v], rows32_v)
            pltpu.sync_copy(rows32_v, so.at[pl.ds(off, CH)])

    for af, atab, po in ((a0f, ta0, p0_o), (a1f, ta1, p1_o)):
        pltpu.sync_copy(zrows, pooled_sh.at[pl.ds(sid * BPW, BPW)])

        @pl.loop(0, N_ACH)
        def _(c, af=af, atab=atab):
            off = base * HIST + c * CH
            pltpu.sync_copy(af.at[pl.ds(off, CH)], idx_v)
            pltpu.sync_copy(atab.at[idx_v], rows32_v)
            pltpu.sync_copy(rows32_v, pooled_sh.at[seg_v.at[c]], add=True)

        pltpu.sync_copy(pooled_sh.at[pl.ds(sid * BPW, BPW)],
                        po.at[pl.ds(base, BPW)])


ROWS = 512


def _mlp_body(uid, sp0, sp1, sp2, sp3, p0, p1, dn, w1, w1d, b1, w2, b2, out):
    acc = jnp.dot(uid[...], w1[0:64, :], preferred_element_type=jnp.float32)
    acc += jnp.dot(sp0[...], w1[64:96, :], preferred_element_type=jnp.float32)
    acc += jnp.dot(sp1[...], w1[96:128, :], preferred_element_type=jnp.float32)
    acc += jnp.dot(sp2[...], w1[128:160, :], preferred_element_type=jnp.float32)
    acc += jnp.dot(sp3[...], w1[160:192, :], preferred_element_type=jnp.float32)
    acc += jnp.dot(p0[...], w1[192:224, :], preferred_element_type=jnp.float32)
    acc += jnp.dot(p1[...], w1[224:256, :], preferred_element_type=jnp.float32)
    acc += jnp.dot(dn[...], w1d[...], preferred_element_type=jnp.float32)
    h = jnp.maximum(acc + b1[...], 0.0)
    out[...] = jnp.dot(h, w2[...], preferred_element_type=jnp.float32) + b2[...]


def _mlp(uid, sp0, sp1, sp2, sp3, p0, p1, dn, w1, w1d, b1, w2, b2):
    grid = (B // ROWS,)
    row_spec = lambda d: pl.BlockSpec((ROWS, d), lambda i: (i, 0))
    full_spec = lambda a, b: pl.BlockSpec((a, b), lambda i: (0, 0))
    return pl.pallas_call(
        _mlp_body,
        grid=grid,
        in_specs=[
            row_spec(D_UID),
            row_spec(D_SP), row_spec(D_SP), row_spec(D_SP), row_spec(D_SP),
            row_spec(D_AR), row_spec(D_AR),
            row_spec(8),
            full_spec(256, DNN_H),
            full_spec(8, DNN_H),
            full_spec(1, DNN_H),
            full_spec(DNN_H, DNN_OUT),
            full_spec(1, DNN_OUT),
        ],
        out_specs=pl.BlockSpec((ROWS, DNN_OUT), lambda i: (i, 0)),
        out_shape=jax.ShapeDtypeStruct((B, DNN_OUT), jnp.float32),
        compiler_params=pltpu.CompilerParams(
            dimension_semantics=("arbitrary",)),
    )(uid, sp0, sp1, sp2, sp3, p0, p1, dn, w1, w1d, b1, w2, b2)


def kernel(seq_id, sparse_0, sparse_1, sparse_2, sparse_3, array_0, array_1,
           dense_0, dense_1, dense_2, emb_user_id,
           emb_sparse_0, emb_sparse_1, emb_sparse_2, emb_sparse_3,
           emb_array_0, emb_array_1, W1, b1, W2, b2):
    a0f = array_0.reshape(-1)
    a1f = array_1.reshape(-1)
    seg_local = (jnp.arange(AR_PER_W, dtype=jnp.int32) // HIST).reshape(
        1, N_ACH, CH)
    seg = seg_local + (jnp.arange(NS, dtype=jnp.int32) * BPW)[:, None, None]
    zrows = jnp.zeros((BPW, D_AR), jnp.float32)
    uid_g, sp0_g, sp1_g, sp2_g, sp3_g, p0, p1 = _sc_gather(
        seq_id, sparse_0, sparse_1, sparse_2, sparse_3, a0f, a1f, seg, zrows,
        emb_user_id, emb_sparse_0, emb_sparse_1, emb_sparse_2, emb_sparse_3,
        emb_array_0, emb_array_1)
    dn = jnp.pad(jnp.stack([dense_0, dense_1, dense_2], axis=1),
                 ((0, 0), (0, 5)))
    w1main = W1[:256]
    w1d = jnp.pad(W1[256:], ((0, 5), (0, 0)))
    return _mlp(uid_g, sp0_g, sp1_g, sp2_g, sp3_g, p0, p1,
                dn, w1main, w1d, b1.reshape(1, -1), W2, b2.reshape(1, -1))
